# baseline (speedup 1.0000x reference)
import functools

import jax
import jax.numpy as jnp
from jax import lax
from jax.experimental import pallas as pl
from jax.experimental.pallas import tpu as pltpu
from jax.experimental.pallas import tpu_sc as plsc

N = 10000
D = 128
E = 160000
R = 3
NC = 2
NS = 16
CHUNK = 128
EPC = E // NC
CPC = EPC // CHUNK
KMAX = -(-CPC // NS)
NP = 10240
ROWS_PER_SUB = NP // NS
ZROWS = 128


def _sc_body(feat_ref, edges_ref, out_ref, acc, zbuf, gbuf, src_idx, dst_idx):
    c = lax.axis_index("c")
    s = lax.axis_index("s")

    @pl.loop(0, ZROWS)
    def _zero_zbuf(i):
        for jj in range(D // 16):
            zbuf[i, pl.ds(jj * 16, 16)] = jnp.zeros((16,), jnp.float32)

    row0 = s * ROWS_PER_SUB
    for r in range(R):
        for z in range(ROWS_PER_SUB // ZROWS):
            pltpu.sync_copy(zbuf, acc.at[pl.ds(row0 + z * ZROWS, ZROWS)])
        plsc.subcore_barrier()

        @pl.loop(0, KMAX)
        def _chunks(k):
            base = (c * (NS * KMAX) + k * NS + s) * CHUNK
            pltpu.sync_copy(edges_ref.at[r, 0, pl.ds(base, CHUNK)], src_idx)
            pltpu.sync_copy(edges_ref.at[r, 1, pl.ds(base, CHUNK)], dst_idx)
            pltpu.sync_copy(feat_ref.at[src_idx], gbuf)
            pltpu.sync_copy(gbuf, acc.at[dst_idx], add=True)

        plsc.subcore_barrier()

        pltpu.sync_copy(acc.at[pl.ds(row0, ROWS_PER_SUB)],
                        out_ref.at[r * NC + c, pl.ds(row0, ROWS_PER_SUB)])
        plsc.subcore_barrier()


_sc_aggregate = functools.partial(
    pl.kernel,
    out_type=jax.ShapeDtypeStruct((R * NC, NP, D), jnp.float32),
    mesh=plsc.VectorSubcoreMesh(
        core_axis_name="c", subcore_axis_name="s",
        num_cores=NC, num_subcores=NS),
    scratch_types=[
        pltpu.VMEM_SHARED((NP, D), jnp.float32),
        pltpu.VMEM((ZROWS, D), jnp.float32),
        pltpu.VMEM((CHUNK, D), jnp.float32),
        pltpu.VMEM((CHUNK,), jnp.int32),
        pltpu.VMEM((CHUNK,), jnp.int32),
    ],
)(_sc_body)


BLK = 1000


def _tc_body(parts_ref, feat_ref, wt_ref, out_ref):
    q0 = parts_ref[0] + parts_ref[1]
    q1 = parts_ref[2] + parts_ref[3]
    q2 = parts_ref[4] + parts_ref[5]
    h = jnp.dot(feat_ref[...], wt_ref[3], preferred_element_type=jnp.float32)
    h = h + jnp.dot(q0, wt_ref[0], preferred_element_type=jnp.float32)
    h = h + jnp.dot(q1, wt_ref[1], preferred_element_type=jnp.float32)
    h = h - jnp.dot(q2, wt_ref[2], preferred_element_type=jnp.float32)
    out_ref[...] = jnp.maximum(h, 0.0)


def _tc_combine(parts, feats, wt):
    return pl.pallas_call(
        _tc_body,
        grid=(N // BLK,),
        in_specs=[
            pl.BlockSpec((R * NC, BLK, D), lambda i: (0, i, 0)),
            pl.BlockSpec((BLK, D), lambda i: (i, 0)),
            pl.BlockSpec((4, D, D), lambda i: (0, 0, 0)),
        ],
        out_specs=pl.BlockSpec((BLK, D), lambda i: (i, 0)),
        out_shape=jax.ShapeDtypeStruct((N, D), jnp.float32),
    )(parts, feats, wt)


EP = NC * NS * KMAX * CHUNK


def kernel(features, W_r0, W_r1, W_r2, W_self, edge_index_r0, edge_index_r1,
           edge_index_r2):
    edges = jnp.stack([edge_index_r0, edge_index_r1, edge_index_r2])
    npad = EP - E
    pad_src = jnp.arange(npad, dtype=jnp.int32) % N
    pad_dst = N + (jnp.arange(npad, dtype=jnp.int32) % (NP - N))
    pad = jnp.stack([pad_src, pad_dst])
    edges = jnp.concatenate([edges, jnp.broadcast_to(pad, (R, 2, npad))],
                            axis=2)
    parts = _sc_aggregate(features, edges)
    wt = jnp.stack([W_r0.T, W_r1.T, W_r2.T, W_self.T])
    return _tc_combine(parts, features, wt)


# speedup vs baseline: 1.5299x; 1.5299x over previous
import functools

import jax
import jax.numpy as jnp
from jax import lax
from jax.experimental import pallas as pl
from jax.experimental.pallas import tpu as pltpu
from jax.experimental.pallas import tpu_sc as plsc

N = 10000
D = 128
E = 160000
R = 3
NC = 2
NS = 16
CHUNK = 128
EPC = E // NC
CPC = EPC // CHUNK
KMAX = -(-CPC // NS)
NP = 10240
ROWS_PER_SUB = NP // NS
ZROWS = 64


def _sc_body(feat_ref, edges_ref, out_ref, acc, zbuf, gbuf0, gbuf1,
             sidx0, sidx1, didx0, didx1, sem0, sem1):
    c = lax.axis_index("c")
    s = lax.axis_index("s")
    gbufs = (gbuf0, gbuf1)
    sidxs = (sidx0, sidx1)
    didxs = (didx0, didx1)
    sems = (sem0, sem1)

    @pl.loop(0, ZROWS)
    def _zero_zbuf(i):
        for jj in range(D // 16):
            zbuf[i, pl.ds(jj * 16, 16)] = jnp.zeros((16,), jnp.float32)

    row0 = s * ROWS_PER_SUB
    for r in range(R):
        for z in range(ROWS_PER_SUB // ZROWS):
            pltpu.sync_copy(zbuf, acc.at[pl.ds(row0 + z * ZROWS, ZROWS)])
        plsc.subcore_barrier()

        cb = c * (NS * KMAX)
        for b in range(2):
            eb = (cb + b * NS + s) * CHUNK
            pltpu.sync_copy(edges_ref.at[r, 0, pl.ds(eb, CHUNK)], sidxs[b])
            pltpu.sync_copy(edges_ref.at[r, 1, pl.ds(eb, CHUNK)], didxs[b])
            pltpu.async_copy(feat_ref.at[sidxs[b]], gbufs[b], sems[b])

        @pl.loop(0, KMAX, step=2)
        def _chunks(k):
            for b in range(2):
                gb, sem = gbufs[b], sems[b]
                pltpu.make_async_copy(feat_ref.at[sidxs[b]], gb, sem).wait()
                pltpu.sync_copy(gb, acc.at[didxs[b]], add=True)

                @pl.when(k + b + 2 < KMAX)
                def _():
                    eb = (cb + (k + b + 2) * NS + s) * CHUNK
                    pltpu.sync_copy(
                        edges_ref.at[r, 0, pl.ds(eb, CHUNK)], sidxs[b])
                    pltpu.sync_copy(
                        edges_ref.at[r, 1, pl.ds(eb, CHUNK)], didxs[b])
                    pltpu.async_copy(feat_ref.at[sidxs[b]], gb, sem)

        plsc.subcore_barrier()

        pltpu.sync_copy(acc.at[pl.ds(row0, ROWS_PER_SUB)],
                        out_ref.at[r * NC + c, pl.ds(row0, ROWS_PER_SUB)])
        plsc.subcore_barrier()


_sc_aggregate = functools.partial(
    pl.kernel,
    out_type=jax.ShapeDtypeStruct((R * NC, NP, D), jnp.float32),
    mesh=plsc.VectorSubcoreMesh(
        core_axis_name="c", subcore_axis_name="s",
        num_cores=NC, num_subcores=NS),
    scratch_types=[
        pltpu.VMEM_SHARED((NP, D), jnp.float32),
        pltpu.VMEM((ZROWS, D), jnp.float32),
        pltpu.VMEM((CHUNK, D), jnp.float32),
        pltpu.VMEM((CHUNK, D), jnp.float32),
        pltpu.VMEM((CHUNK,), jnp.int32),
        pltpu.VMEM((CHUNK,), jnp.int32),
        pltpu.VMEM((CHUNK,), jnp.int32),
        pltpu.VMEM((CHUNK,), jnp.int32),
        pltpu.SemaphoreType.DMA,
        pltpu.SemaphoreType.DMA,
    ],
)(_sc_body)


BLK = 1000


def _tc_body(parts_ref, feat_ref, wt_ref, out_ref):
    q0 = parts_ref[0] + parts_ref[1]
    q1 = parts_ref[2] + parts_ref[3]
    q2 = parts_ref[4] + parts_ref[5]
    h = jnp.dot(feat_ref[...], wt_ref[3], preferred_element_type=jnp.float32)
    h = h + jnp.dot(q0, wt_ref[0], preferred_element_type=jnp.float32)
    h = h + jnp.dot(q1, wt_ref[1], preferred_element_type=jnp.float32)
    h = h - jnp.dot(q2, wt_ref[2], preferred_element_type=jnp.float32)
    out_ref[...] = jnp.maximum(h, 0.0)


def _tc_combine(parts, feats, wt):
    return pl.pallas_call(
        _tc_body,
        grid=(N // BLK,),
        in_specs=[
            pl.BlockSpec((R * NC, BLK, D), lambda i: (0, i, 0)),
            pl.BlockSpec((BLK, D), lambda i: (i, 0)),
            pl.BlockSpec((4, D, D), lambda i: (0, 0, 0)),
        ],
        out_specs=pl.BlockSpec((BLK, D), lambda i: (i, 0)),
        out_shape=jax.ShapeDtypeStruct((N, D), jnp.float32),
    )(parts, feats, wt)


EP = NC * NS * KMAX * CHUNK


def kernel(features, W_r0, W_r1, W_r2, W_self, edge_index_r0, edge_index_r1,
           edge_index_r2):
    edges = jnp.stack([edge_index_r0, edge_index_r1, edge_index_r2])
    npad = EP - E
    pad_src = jnp.arange(npad, dtype=jnp.int32) % N
    pad_dst = N + (jnp.arange(npad, dtype=jnp.int32) % (NP - N))
    pad = jnp.stack([pad_src, pad_dst])
    edges = jnp.concatenate([edges, jnp.broadcast_to(pad, (R, 2, npad))],
                            axis=2)
    parts = _sc_aggregate(features, edges)
    wt = jnp.stack([W_r0.T, W_r1.T, W_r2.T, W_self.T])
    return _tc_combine(parts, features, wt)


# speedup vs baseline: 1.7113x; 1.1185x over previous
import functools

import jax
import jax.numpy as jnp
from jax import lax
from jax.experimental import pallas as pl
from jax.experimental.pallas import tpu as pltpu
from jax.experimental.pallas import tpu_sc as plsc

N = 10000
D = 128
E = 160000
R = 3
NC = 2
NS = 16
CHUNK = 128
NCH = E // CHUNK
T = NCH // (NC * NS)
XTRA = NCH - NC * NS * T
NP = 10112
ROWS_PER_SUB = NP // NS


def _sc_body(feat_ref, edges_ref, zeros_ref, out_ref, acc,
             gb0, gb1, gb2, si0, si1, si2, di0, di1, di2,
             gs0, gs1, gs2, ss0, ss1, ss2):
    c = lax.axis_index("c")
    s = lax.axis_index("s")
    w = c * NS + s
    start = w * T + jnp.minimum(w, XTRA)
    row0 = s * ROWS_PER_SUB
    gbufs = (gb0, gb1, gb2)
    sidxs = (si0, si1, si2)
    didxs = (di0, di1, di2)
    gsems = (gs0, gs1, gs2)
    ssems = (ss0, ss1, ss2)

    for r in range(R):
        pltpu.sync_copy(zeros_ref, acc.at[pl.ds(row0, ROWS_PER_SUB)])
        plsc.subcore_barrier()

        def idx_load(i, b, r=r):
            eb = (start + i) * CHUNK
            pltpu.sync_copy(edges_ref.at[r, 0, pl.ds(eb, CHUNK)], sidxs[b])
            pltpu.sync_copy(edges_ref.at[r, 1, pl.ds(eb, CHUNK)], didxs[b])

        for b in range(2):
            idx_load(b, b)
            pltpu.async_copy(feat_ref.at[sidxs[b]], gbufs[b], gsems[b])

        @pl.loop(0, T, step=3)
        def _chunks(k):
            for b in range(3):
                j = k + b
                b2 = (b + 2) % 3
                pltpu.make_async_copy(
                    feat_ref.at[sidxs[b]], gbufs[b], gsems[b]).wait()
                pltpu.async_copy(
                    gbufs[b], acc.at[didxs[b]], ssems[b], add=True)

                @pl.when(j + 2 < T)
                def _():
                    @pl.when(j >= 1)
                    def _():
                        pltpu.make_async_copy(
                            gbufs[b2], acc.at[didxs[b2]], ssems[b2]).wait()
                    idx_load(j + 2, b2)
                    pltpu.async_copy(
                        feat_ref.at[sidxs[b2]], gbufs[b2], gsems[b2])

        for b in range(3):
            pltpu.make_async_copy(gbufs[b], acc.at[didxs[b]], ssems[b]).wait()

        @pl.when(w < XTRA)
        def _():
            eb = (NC * NS * T + w) * CHUNK
            pltpu.sync_copy(edges_ref.at[r, 0, pl.ds(eb, CHUNK)], si0)
            pltpu.sync_copy(edges_ref.at[r, 1, pl.ds(eb, CHUNK)], di0)
            pltpu.sync_copy(feat_ref.at[si0], gb0)
            pltpu.sync_copy(gb0, acc.at[di0], add=True)

        plsc.subcore_barrier()

        pltpu.sync_copy(acc.at[pl.ds(row0, ROWS_PER_SUB)],
                        out_ref.at[r * NC + c, pl.ds(row0, ROWS_PER_SUB)])
        plsc.subcore_barrier()


_sc_aggregate = functools.partial(
    pl.kernel,
    out_type=jax.ShapeDtypeStruct((R * NC, NP, D), jnp.float32),
    mesh=plsc.VectorSubcoreMesh(
        core_axis_name="c", subcore_axis_name="s",
        num_cores=NC, num_subcores=NS),
    scratch_types=[
        pltpu.VMEM_SHARED((NP, D), jnp.float32),
        pltpu.VMEM((CHUNK, D), jnp.float32),
        pltpu.VMEM((CHUNK, D), jnp.float32),
        pltpu.VMEM((CHUNK, D), jnp.float32),
        pltpu.VMEM((CHUNK,), jnp.int32),
        pltpu.VMEM((CHUNK,), jnp.int32),
        pltpu.VMEM((CHUNK,), jnp.int32),
        pltpu.VMEM((CHUNK,), jnp.int32),
        pltpu.VMEM((CHUNK,), jnp.int32),
        pltpu.VMEM((CHUNK,), jnp.int32),
        pltpu.SemaphoreType.DMA,
        pltpu.SemaphoreType.DMA,
        pltpu.SemaphoreType.DMA,
        pltpu.SemaphoreType.DMA,
        pltpu.SemaphoreType.DMA,
        pltpu.SemaphoreType.DMA,
    ],
)(_sc_body)


BLK = 1000


def _tc_body(parts_ref, feat_ref, wt_ref, out_ref):
    q0 = parts_ref[0] + parts_ref[1]
    q1 = parts_ref[2] + parts_ref[3]
    q2 = parts_ref[4] + parts_ref[5]
    h = jnp.dot(feat_ref[...], wt_ref[3], preferred_element_type=jnp.float32)
    h = h + jnp.dot(q0, wt_ref[0], preferred_element_type=jnp.float32)
    h = h + jnp.dot(q1, wt_ref[1], preferred_element_type=jnp.float32)
    h = h - jnp.dot(q2, wt_ref[2], preferred_element_type=jnp.float32)
    out_ref[...] = jnp.maximum(h, 0.0)


def _tc_combine(parts, feats, wt):
    return pl.pallas_call(
        _tc_body,
        grid=(N // BLK,),
        in_specs=[
            pl.BlockSpec((R * NC, BLK, D), lambda i: (0, i, 0)),
            pl.BlockSpec((BLK, D), lambda i: (i, 0)),
            pl.BlockSpec((4, D, D), lambda i: (0, 0, 0)),
        ],
        out_specs=pl.BlockSpec((BLK, D), lambda i: (i, 0)),
        out_shape=jax.ShapeDtypeStruct((N, D), jnp.float32),
    )(parts, feats, wt)


def kernel(features, W_r0, W_r1, W_r2, W_self, edge_index_r0, edge_index_r1,
           edge_index_r2):
    edges = jnp.stack([edge_index_r0, edge_index_r1, edge_index_r2])
    zeros = jnp.zeros((ROWS_PER_SUB, D), jnp.float32)
    parts = _sc_aggregate(features, edges, zeros)
    wt = jnp.stack([W_r0.T, W_r1.T, W_r2.T, W_self.T])
    return _tc_combine(parts, features, wt)


# speedup vs baseline: 1.7141x; 1.0017x over previous
import functools

import jax
import jax.numpy as jnp
from jax import lax
from jax.experimental import pallas as pl
from jax.experimental.pallas import tpu as pltpu
from jax.experimental.pallas import tpu_sc as plsc

N = 10000
D = 128
E = 160000
R = 3
NC = 2
NS = 16
CHUNK = 128
NCH = E // CHUNK
T = NCH // (NC * NS)
XTRA = NCH - NC * NS * T
NP = 10112
ROWS_PER_SUB = NP // NS


def _sc_body(feat_ref, edges_ref, zeros_ref, out_ref, acc,
             gb0, gb1, gb2, si0, si1, si2, di0, di1, di2,
             gs0, gs1, gs2, ss0, ss1, ss2):
    c = lax.axis_index("c")
    s = lax.axis_index("s")
    w = c * NS + s
    start = w * T + jnp.minimum(w, XTRA)
    row0 = s * ROWS_PER_SUB
    gbufs = (gb0, gb1, gb2)
    sidxs = (si0, si1, si2)
    didxs = (di0, di1, di2)
    gsems = (gs0, gs1, gs2)
    ssems = (ss0, ss1, ss2)

    for r in range(R):
        pltpu.sync_copy(zeros_ref, acc.at[pl.ds(row0, ROWS_PER_SUB)])
        plsc.subcore_barrier()

        def idx_load(i, b, r=r):
            eb = (start + i) * CHUNK
            pltpu.sync_copy(edges_ref.at[r, 0, pl.ds(eb, CHUNK)], sidxs[b])
            pltpu.sync_copy(edges_ref.at[r, 1, pl.ds(eb, CHUNK)], didxs[b])

        for b in range(2):
            idx_load(b, b)
            pltpu.async_copy(feat_ref.at[sidxs[b]], gbufs[b], gsems[b])

        @pl.loop(0, T, step=3)
        def _chunks(k):
            for b in range(3):
                j = k + b
                b2 = (b + 2) % 3
                pltpu.make_async_copy(
                    feat_ref.at[sidxs[b]], gbufs[b], gsems[b]).wait()
                pltpu.async_copy(
                    gbufs[b], acc.at[didxs[b]], ssems[b], add=True)

                @pl.when(j + 2 < T)
                def _():
                    @pl.when(j >= 1)
                    def _():
                        pltpu.make_async_copy(
                            gbufs[b2], acc.at[didxs[b2]], ssems[b2]).wait()
                    idx_load(j + 2, b2)
                    pltpu.async_copy(
                        feat_ref.at[sidxs[b2]], gbufs[b2], gsems[b2])

        for b in range(3):
            pltpu.make_async_copy(gbufs[b], acc.at[didxs[b]], ssems[b]).wait()

        @pl.when(w < XTRA)
        def _():
            eb = (start + T) * CHUNK
            pltpu.sync_copy(edges_ref.at[r, 0, pl.ds(eb, CHUNK)], si0)
            pltpu.sync_copy(edges_ref.at[r, 1, pl.ds(eb, CHUNK)], di0)
            pltpu.sync_copy(feat_ref.at[si0], gb0)
            pltpu.sync_copy(gb0, acc.at[di0], add=True)

        plsc.subcore_barrier()

        pltpu.sync_copy(acc.at[pl.ds(row0, ROWS_PER_SUB)],
                        out_ref.at[r * NC + c, pl.ds(row0, ROWS_PER_SUB)])
        plsc.subcore_barrier()


_sc_aggregate = functools.partial(
    pl.kernel,
    out_type=jax.ShapeDtypeStruct((R * NC, NP, D), jnp.float32),
    mesh=plsc.VectorSubcoreMesh(
        core_axis_name="c", subcore_axis_name="s",
        num_cores=NC, num_subcores=NS),
    scratch_types=[
        pltpu.VMEM_SHARED((NP, D), jnp.float32),
        pltpu.VMEM((CHUNK, D), jnp.float32),
        pltpu.VMEM((CHUNK, D), jnp.float32),
        pltpu.VMEM((CHUNK, D), jnp.float32),
        pltpu.VMEM((CHUNK,), jnp.int32),
        pltpu.VMEM((CHUNK,), jnp.int32),
        pltpu.VMEM((CHUNK,), jnp.int32),
        pltpu.VMEM((CHUNK,), jnp.int32),
        pltpu.VMEM((CHUNK,), jnp.int32),
        pltpu.VMEM((CHUNK,), jnp.int32),
        pltpu.SemaphoreType.DMA,
        pltpu.SemaphoreType.DMA,
        pltpu.SemaphoreType.DMA,
        pltpu.SemaphoreType.DMA,
        pltpu.SemaphoreType.DMA,
        pltpu.SemaphoreType.DMA,
    ],
)(_sc_body)


BLK = 1000


def _tc_body(parts_ref, feat_ref, wt_ref, out_ref):
    q0 = parts_ref[0] + parts_ref[1]
    q1 = parts_ref[2] + parts_ref[3]
    q2 = parts_ref[4] + parts_ref[5]
    h = jnp.dot(feat_ref[...], wt_ref[3], preferred_element_type=jnp.float32)
    h = h + jnp.dot(q0, wt_ref[0], preferred_element_type=jnp.float32)
    h = h + jnp.dot(q1, wt_ref[1], preferred_element_type=jnp.float32)
    h = h - jnp.dot(q2, wt_ref[2], preferred_element_type=jnp.float32)
    out_ref[...] = jnp.maximum(h, 0.0)


def _tc_combine(parts, feats, wt):
    return pl.pallas_call(
        _tc_body,
        grid=(N // BLK,),
        in_specs=[
            pl.BlockSpec((R * NC, BLK, D), lambda i: (0, i, 0)),
            pl.BlockSpec((BLK, D), lambda i: (i, 0)),
            pl.BlockSpec((4, D, D), lambda i: (0, 0, 0)),
        ],
        out_specs=pl.BlockSpec((BLK, D), lambda i: (i, 0)),
        out_shape=jax.ShapeDtypeStruct((N, D), jnp.float32),
    )(parts, feats, wt)


def kernel(features, W_r0, W_r1, W_r2, W_self, edge_index_r0, edge_index_r1,
           edge_index_r2):
    edges = jnp.stack([edge_index_r0, edge_index_r1, edge_index_r2])
    zeros = jnp.zeros((ROWS_PER_SUB, D), jnp.float32)
    parts = _sc_aggregate(features, edges, zeros)
    wt = jnp.stack([W_r0.T, W_r1.T, W_r2.T, W_self.T])
    return _tc_combine(parts, features, wt)


# speedup vs baseline: 1.7311x; 1.0099x over previous
import functools

import jax
import jax.numpy as jnp
from jax import lax
from jax.experimental import pallas as pl
from jax.experimental.pallas import tpu as pltpu
from jax.experimental.pallas import tpu_sc as plsc

N = 10000
D = 128
E = 160000
R = 3
NC = 2
NS = 16
CHUNK = 128
NCH = E // CHUNK
T = NCH // (NC * NS)
XTRA = NCH - NC * NS * T
NP = 10112
ROWS_PER_SUB = NP // NS


def _sc_body(feat_ref, edges_ref, zeros_ref, out_ref, acc,
             gb0, gb1, gb2, si0, si1, si2, di0, di1, di2,
             gs0, gs1, gs2, ss0, ss1, ss2):
    c = lax.axis_index("c")
    s = lax.axis_index("s")
    w = c * NS + s
    start = w * T + jnp.minimum(w, XTRA)
    row0 = s * ROWS_PER_SUB
    gbufs = (gb0, gb1, gb2)
    sidxs = (si0, si1, si2)
    didxs = (di0, di1, di2)
    gsems = (gs0, gs1, gs2)
    ssems = (ss0, ss1, ss2)

    for r in range(R):
        pltpu.sync_copy(zeros_ref, acc.at[pl.ds(row0, ROWS_PER_SUB)])
        plsc.subcore_barrier()

        def idx_load(i, b, r=r):
            eb = (start + i) * CHUNK
            pltpu.sync_copy(edges_ref.at[r, 0, pl.ds(eb, CHUNK)], sidxs[b])
            pltpu.sync_copy(edges_ref.at[r, 1, pl.ds(eb, CHUNK)], didxs[b])

        for b in range(2):
            idx_load(b, b)
            pltpu.async_copy(feat_ref.at[sidxs[b]], gbufs[b], gsems[b])

        @pl.loop(0, T, step=3)
        def _chunks(k):
            for b in range(3):
                j = k + b
                b2 = (b + 2) % 3
                pltpu.make_async_copy(
                    feat_ref.at[sidxs[b]], gbufs[b], gsems[b]).wait()
                pltpu.async_copy(
                    gbufs[b], acc.at[didxs[b]], ssems[b], add=True)

                @pl.when(j + 2 < T)
                def _():
                    @pl.when(j >= 1)
                    def _():
                        pltpu.make_async_copy(
                            gbufs[b2], acc.at[didxs[b2]], ssems[b2]).wait()
                    idx_load(j + 2, b2)
                    pltpu.async_copy(
                        feat_ref.at[sidxs[b2]], gbufs[b2], gsems[b2])

        for b in range(3):
            pltpu.make_async_copy(gbufs[b], acc.at[didxs[b]], ssems[b]).wait()

        @pl.when(w < XTRA)
        def _():
            eb = (start + T) * CHUNK
            pltpu.sync_copy(edges_ref.at[r, 0, pl.ds(eb, CHUNK)], si0)
            pltpu.sync_copy(edges_ref.at[r, 1, pl.ds(eb, CHUNK)], di0)
            pltpu.sync_copy(feat_ref.at[si0], gb0)
            pltpu.sync_copy(gb0, acc.at[di0], add=True)

        plsc.subcore_barrier()

        pltpu.sync_copy(acc.at[pl.ds(row0, ROWS_PER_SUB)],
                        out_ref.at[r * NC + c, pl.ds(row0, ROWS_PER_SUB)])
        plsc.subcore_barrier()


_sc_aggregate = functools.partial(
    pl.kernel,
    out_type=jax.ShapeDtypeStruct((R * NC, NP, D), jnp.float32),
    mesh=plsc.VectorSubcoreMesh(
        core_axis_name="c", subcore_axis_name="s",
        num_cores=NC, num_subcores=NS),
    scratch_types=[
        pltpu.VMEM_SHARED((NP, D), jnp.float32),
        pltpu.VMEM((CHUNK, D), jnp.float32),
        pltpu.VMEM((CHUNK, D), jnp.float32),
        pltpu.VMEM((CHUNK, D), jnp.float32),
        pltpu.VMEM((CHUNK,), jnp.int32),
        pltpu.VMEM((CHUNK,), jnp.int32),
        pltpu.VMEM((CHUNK,), jnp.int32),
        pltpu.VMEM((CHUNK,), jnp.int32),
        pltpu.VMEM((CHUNK,), jnp.int32),
        pltpu.VMEM((CHUNK,), jnp.int32),
        pltpu.SemaphoreType.DMA,
        pltpu.SemaphoreType.DMA,
        pltpu.SemaphoreType.DMA,
        pltpu.SemaphoreType.DMA,
        pltpu.SemaphoreType.DMA,
        pltpu.SemaphoreType.DMA,
    ],
)(_sc_body)


BLK = 2000


def _tc_body(parts_ref, feat_ref, wt_ref, out_ref):
    q0 = parts_ref[0] + parts_ref[1]
    q1 = parts_ref[2] + parts_ref[3]
    q2 = parts_ref[4] + parts_ref[5]
    h = jnp.dot(feat_ref[...], wt_ref[3], preferred_element_type=jnp.float32)
    h = h + jnp.dot(q0, wt_ref[0], preferred_element_type=jnp.float32)
    h = h + jnp.dot(q1, wt_ref[1], preferred_element_type=jnp.float32)
    h = h - jnp.dot(q2, wt_ref[2], preferred_element_type=jnp.float32)
    out_ref[...] = jnp.maximum(h, 0.0)


def _tc_combine(parts, feats, wt):
    return pl.pallas_call(
        _tc_body,
        grid=(N // BLK,),
        in_specs=[
            pl.BlockSpec((R * NC, BLK, D), lambda i: (0, i, 0)),
            pl.BlockSpec((BLK, D), lambda i: (i, 0)),
            pl.BlockSpec((4, D, D), lambda i: (0, 0, 0)),
        ],
        out_specs=pl.BlockSpec((BLK, D), lambda i: (i, 0)),
        out_shape=jax.ShapeDtypeStruct((N, D), jnp.float32),
    )(parts, feats, wt)


def kernel(features, W_r0, W_r1, W_r2, W_self, edge_index_r0, edge_index_r1,
           edge_index_r2):
    edges = jnp.stack([edge_index_r0, edge_index_r1, edge_index_r2])
    zeros = jnp.zeros((ROWS_PER_SUB, D), jnp.float32)
    parts = _sc_aggregate(features, edges, zeros)
    wt = jnp.stack([W_r0.T, W_r1.T, W_r2.T, W_self.T])
    return _tc_combine(parts, features, wt)
